# baseline (speedup 1.0000x reference)
import functools

import jax
import jax.numpy as jnp
from jax import lax
from jax.experimental import pallas as pl
from jax.experimental.pallas import tpu as pltpu
from jax.experimental.pallas import tpu_sc as plsc

VOCAB = 100000
DIM = 64
B = 16384
L = 200

NC = 2
NS = 16
NW = NC * NS
ROWS_PER_W = B // NW
C = 8
CHUNKS = ROWS_PER_W // C
SPLIT = (104, 96)
NG = 2 * C
NIDX = C * L
LANES = 16
NG32 = DIM // 32
UNROLL = 4


def _sum_rows(rows, base):
    zeros = jnp.zeros((LANES,), jnp.float32)

    def body(t, accs):
        r = base + t * UNROLL
        a = list(accs)
        for u in range(UNROLL):
            for k in range(NG32):
                packed = plsc.bitcast(
                    rows[r + u, pl.ds(k * 16, 16)], jnp.bfloat16
                )
                lo, hi = plsc.unpack(packed, format=plsc.PackFormat.INTERLEAVED)
                a[2 * k] = a[2 * k] + lo
                a[2 * k + 1] = a[2 * k + 1] + hi
        return tuple(a)

    return lax.fori_loop(0, L // UNROLL, body, (zeros,) * 4)


def _worker(x_hbm, table_hbm, out_hbm, idx_v, rows_v, acc_v,
            isem0, isem1, gsem0, gsem1):
    wid = lax.axis_index("s") * NC + lax.axis_index("c")
    base_row = wid * ROWS_PER_W
    isems = (isem0, isem1)
    gsems = (gsem0, gsem1)
    even = 2 * lax.iota(jnp.int32, LANES)
    odd = even + 1

    def idx_start(g, b):
        r0 = (base_row + g * C) * L
        pltpu.async_copy(x_hbm.at[pl.ds(r0, NIDX)], idx_v.at[b], isems[b])

    def idx_wait(b):
        pltpu.make_async_copy(
            x_hbm.at[pl.ds(0, NIDX)], idx_v.at[b], isems[b]
        ).wait()

    def gathers_start(b):
        for i in range(C):
            off = 0
            for n in SPLIT:
                pltpu.async_copy(
                    table_hbm.at[idx_v.at[b, pl.ds(i * L + off, n)]],
                    rows_v.at[b, pl.ds(i * L + off, n)],
                    gsems[b],
                )
                off += n

    def gathers_drain(b):
        pltpu.make_async_copy(
            table_hbm.at[pl.ds(0, NIDX)], rows_v.at[b], gsems[b]
        ).wait()

    def consume(g, b):
        for i in range(C):
            accs = _sum_rows(rows_v.at[b], i * L)
            for k in range(NG32):
                base = i * DIM + k * 32
                plsc.store_scatter(acc_v, [base + even], accs[2 * k])
                plsc.store_scatter(acc_v, [base + odd], accs[2 * k + 1])
        pltpu.sync_copy(
            acc_v, out_hbm.at[pl.ds((base_row + g * C) * DIM, C * DIM)]
        )

    idx_start(0, 0)
    idx_start(1, 1)
    idx_wait(0)
    gathers_start(0)

    def pair_body(h, carry):
        g = 2 * h
        idx_wait(1)
        gathers_start(1)
        gathers_drain(0)
        idx_start(g + 2, 0)
        consume(g, 0)
        idx_wait(0)
        gathers_start(0)
        gathers_drain(1)
        idx_start(g + 3, 1)
        consume(g + 1, 1)
        return carry

    lax.fori_loop(0, CHUNKS // 2 - 1, pair_body, 0)

    idx_wait(1)
    gathers_start(1)
    gathers_drain(0)
    consume(CHUNKS - 2, 0)
    gathers_drain(1)
    consume(CHUNKS - 1, 1)


@jax.jit
def _pooled_lookup(x, table):
    x1 = x.reshape(B * L)
    table_bf16 = jax.lax.bitcast_convert_type(
        table.astype(jnp.bfloat16).reshape(VOCAB, DIM // 2, 2), jnp.uint32
    )
    mesh = plsc.VectorSubcoreMesh(core_axis_name="c", subcore_axis_name="s")
    out = pl.kernel(
        _worker,
        mesh=mesh,
        compiler_params=pltpu.CompilerParams(
            use_tc_tiling_on_sc=False, needs_layout_passes=False
        ),
        out_type=jax.ShapeDtypeStruct((B * DIM,), jnp.float32),
        scratch_types=[
            pltpu.VMEM((2, NIDX), jnp.int32),
            pltpu.VMEM((2, NIDX, DIM // 2), jnp.uint32),
            pltpu.VMEM((C * DIM,), jnp.float32),
            pltpu.SemaphoreType.DMA,
            pltpu.SemaphoreType.DMA,
            pltpu.SemaphoreType.DMA,
            pltpu.SemaphoreType.DMA,
        ],
    )(x1, table_bf16)
    return out.reshape(B, DIM)


def kernel(x, table):
    return _pooled_lookup(x, table)


# speedup vs baseline: 1.5040x; 1.5040x over previous
import functools

import jax
import jax.numpy as jnp
from jax import lax
from jax.experimental import pallas as pl
from jax.experimental.pallas import tpu as pltpu
from jax.experimental.pallas import tpu_sc as plsc

VOCAB = 100000
DIM = 64
B = 16384
L = 200

NC = 2
NS = 16
NW = NC * NS
ROWS_PER_W = B // NW
C = 8
CHUNKS = ROWS_PER_W // C
SPLIT = (104, 96)
NG = 2 * C
NIDX = C * L
LANES = 16
NG32 = DIM // 32
UNROLL = 4


def _sum_rows(rows, base):
    zeros = jnp.zeros((LANES,), jnp.float32)

    def body(t, accs):
        r = base + t * UNROLL
        a = list(accs)
        for u in range(UNROLL):
            for k in range(NG32):
                packed = rows[r + u, pl.ds(k * 32, 32)]
                lo, hi = plsc.unpack(packed, format=plsc.PackFormat.INTERLEAVED)
                a[2 * k] = a[2 * k] + lo
                a[2 * k + 1] = a[2 * k + 1] + hi
        return tuple(a)

    return lax.fori_loop(0, L // UNROLL, body, (zeros,) * 4)


def _worker(x_hbm, table_hbm, out_hbm, idx_v, rows_v, acc_v,
            isem0, isem1, gsem0, gsem1):
    wid = lax.axis_index("s") * NC + lax.axis_index("c")
    base_row = wid * ROWS_PER_W
    isems = (isem0, isem1)
    gsems = (gsem0, gsem1)
    even = 2 * lax.iota(jnp.int32, LANES)
    odd = even + 1

    def idx_start(g, b):
        r0 = (base_row + g * C) * L
        pltpu.async_copy(x_hbm.at[pl.ds(r0, NIDX)], idx_v.at[b], isems[b])

    def idx_wait(b):
        pltpu.make_async_copy(
            x_hbm.at[pl.ds(0, NIDX)], idx_v.at[b], isems[b]
        ).wait()

    def gathers_start(b):
        for i in range(C):
            off = 0
            for n in SPLIT:
                pltpu.async_copy(
                    table_hbm.at[idx_v.at[b, pl.ds(i * L + off, n)]],
                    rows_v.at[b, pl.ds(i * L + off, n)],
                    gsems[b],
                )
                off += n

    def gathers_drain(b):
        pltpu.make_async_copy(
            table_hbm.at[pl.ds(0, NIDX)], rows_v.at[b], gsems[b]
        ).wait()

    def consume(g, b):
        for i in range(C):
            accs = _sum_rows(rows_v.at[b], i * L)
            for k in range(NG32):
                base = i * DIM + k * 32
                plsc.store_scatter(acc_v, [base + even], accs[2 * k])
                plsc.store_scatter(acc_v, [base + odd], accs[2 * k + 1])
        pltpu.sync_copy(
            acc_v, out_hbm.at[pl.ds((base_row + g * C) * DIM, C * DIM)]
        )

    idx_start(0, 0)
    idx_start(1, 1)
    idx_wait(0)
    gathers_start(0)

    def pair_body(h, carry):
        g = 2 * h
        idx_wait(1)
        gathers_start(1)
        gathers_drain(0)
        idx_start(g + 2, 0)
        consume(g, 0)
        idx_wait(0)
        gathers_start(0)
        gathers_drain(1)
        idx_start(g + 3, 1)
        consume(g + 1, 1)
        return carry

    lax.fori_loop(0, CHUNKS // 2 - 1, pair_body, 0)

    idx_wait(1)
    gathers_start(1)
    gathers_drain(0)
    consume(CHUNKS - 2, 0)
    gathers_drain(1)
    consume(CHUNKS - 1, 1)


@jax.jit
def _pooled_lookup(x, table):
    x1 = x.reshape(B * L)
    table_bf16 = table.astype(jnp.bfloat16)
    mesh = plsc.VectorSubcoreMesh(core_axis_name="c", subcore_axis_name="s")
    out = pl.kernel(
        _worker,
        mesh=mesh,
        compiler_params=pltpu.CompilerParams(
            use_tc_tiling_on_sc=False, needs_layout_passes=False
        ),
        out_type=jax.ShapeDtypeStruct((B * DIM,), jnp.float32),
        scratch_types=[
            pltpu.VMEM((2, NIDX), jnp.int32),
            pltpu.VMEM((2, NIDX, DIM), jnp.bfloat16),
            pltpu.VMEM((C * DIM,), jnp.float32),
            pltpu.SemaphoreType.DMA,
            pltpu.SemaphoreType.DMA,
            pltpu.SemaphoreType.DMA,
            pltpu.SemaphoreType.DMA,
        ],
    )(x1, table_bf16)
    return out.reshape(B, DIM)


def kernel(x, table):
    return _pooled_lookup(x, table)
